# baseline (speedup 1.0000x reference)
import functools

import jax
import jax.numpy as jnp
from jax import lax
from jax.experimental import pallas as pl
from jax.experimental.pallas import tpu as pltpu
from jax.experimental.pallas import tpu_sc as plsc

_K = 64
_B = 4
_S = 4096
_C = 1024
_MININT = -(2**31)

_NC = 2
_NS = 16
_NW = _NC * _NS
_CPW = _C // (_NW // _B)
_GRP = _CPW // 16


def _tc_threshold_body(x_ref, thr_ref, neq_ref):
    x = x_ref[0]
    b = lax.bitcast_convert_type(x, jnp.int32)
    s = jnp.bitwise_xor(
        b, lax.shift_right_logical(jnp.right_shift(b, 31), jnp.int32(1)))

    hi = jnp.right_shift(s, 16).astype(jnp.int16).reshape(64, 64, 128)
    l16 = ((s & 0xFFFF) - 32768).astype(jnp.int16).reshape(64, 64, 128)

    def _cnt16(mask):
        vals = [mask[a].astype(jnp.int16) for a in range(64)]
        while len(vals) > 1:
            vals = [vals[i] + vals[i + 1] for i in range(0, len(vals), 2)]
        return jnp.sum(vals[0].astype(jnp.int32), axis=0, keepdims=True)

    def _descend(data, k_tgt):
        def bit_step(i, ph):
            bit = jnp.left_shift(jnp.int32(1), 15 - i)
            cand = jnp.bitwise_or(ph, bit)
            cand_s = (cand - 32768).astype(jnp.int16).reshape(1, 1, 128)
            cnt = _cnt16(data >= cand_s)
            return jnp.where(cnt >= k_tgt, cand, ph)

        return lax.fori_loop(0, 16, bit_step, jnp.zeros((1, 128), jnp.int32))

    ph = _descend(hi, _K)
    thi_s = (ph - 32768).astype(jnp.int16).reshape(1, 1, 128)
    n_hi_gt = _cnt16(hi > thi_s)
    k2 = _K - n_hi_gt
    loe = jnp.where(hi == thi_s, l16, jnp.int16(-32768))
    pl_ = _descend(loe, k2)

    t_s = jnp.left_shift(ph - 32768, 16) | pl_
    tlo_s = (pl_ - 32768).astype(jnp.int16).reshape(1, 1, 128)
    n_gt = n_hi_gt + _cnt16(loe > tlo_s)
    b_thr = jnp.where(t_s < 0, jnp.bitwise_not(jnp.bitwise_xor(t_s, _MININT)), t_s)
    thr_ref[...] = lax.bitcast_convert_type(b_thr, jnp.float32)[None, None]
    neq_ref[...] = (_K - n_gt)[None, None]


_CHUNK = 256


def _sc_select_body(x_hbm, thr_hbm, neq_hbm, out_hbm, xbufa, xbufb, obufs,
                    tbuf, nbuf, sema, semb):
    wid = lax.axis_index("c") * _NS + lax.axis_index("s")
    b = wid // (_NW // _B)
    cbase = (wid % (_NW // _B)) * _CPW
    coff = b * _C + cbase
    lane = lax.iota(jnp.int32, 16)

    pltpu.sync_copy(thr_hbm.at[pl.ds(coff, _CPW)], tbuf)
    pltpu.sync_copy(neq_hbm.at[pl.ds(coff, _CPW)], nbuf)

    thrs = [tbuf[pl.ds(g * 16, 16)] for g in range(_GRP)]
    st0 = tuple(
        [lane * _K for _ in range(_GRP)]
        + [nbuf[pl.ds(g * 16, 16)] for g in range(_GRP)])

    def _copy(ch, buf, sem):
        return pltpu.make_async_copy(
            x_hbm.at[b, pl.ds(ch * _CHUNK, _CHUNK), pl.ds(cbase, _CPW)],
            buf, sem)

    def _process(xbuf, st):
        @plsc.parallel_loop(0, _CHUNK, unroll=8, carry=st)
        def step(i, s2):
            vs = [xbuf[i, pl.ds(g * 16, 16)] for g in range(_GRP)]
            addrs, rems = list(s2[:_GRP]), list(s2[_GRP:])
            for g in range(_GRP):
                gt = vs[g] > thrs[g]
                eq = vs[g] == thrs[g]
                sel = jnp.logical_or(gt, jnp.logical_and(eq, rems[g] > 0))
                plsc.store_scatter(obufs[g], [addrs[g]], vs[g], mask=sel)
                rems[g] = rems[g] - eq.astype(jnp.int32)
                addrs[g] = addrs[g] + sel.astype(jnp.int32)
            return tuple(addrs + rems)

        return step

    npairs = _S // _CHUNK // 2
    _copy(0, xbufa, sema).start()

    def pair(p, st):
        _copy(2 * p, xbufa, sema).wait()
        _copy(2 * p + 1, xbufb, semb).start()
        st = _process(xbufa, st)
        _copy(2 * p + 1, xbufb, semb).wait()

        @pl.when(p < npairs - 1)
        def _():
            _copy(2 * p + 2, xbufa, sema).start()

        return _process(xbufb, st)

    lax.fori_loop(0, npairs, pair, st0)
    for g in range(_GRP):
        pltpu.sync_copy(
            obufs[g],
            out_hbm.at[pl.ds(wid * (_CPW * _K) + g * 16 * _K, 16 * _K)])


@functools.cache
def _sc_select():
    return pl.kernel(
        _sc_select_body,
        out_type=jax.ShapeDtypeStruct((_NW * _CPW * _K,), jnp.float32),
        mesh=plsc.VectorSubcoreMesh(
            core_axis_name="c", subcore_axis_name="s",
            num_cores=_NC, num_subcores=_NS,
        ),
        compiler_params=pltpu.CompilerParams(needs_layout_passes=False),
        scratch_types=[
            pltpu.VMEM((_CHUNK, _CPW), jnp.float32),
            pltpu.VMEM((_CHUNK, _CPW), jnp.float32),
            [pltpu.VMEM((16 * _K,), jnp.float32) for _ in range(_GRP)],
            pltpu.VMEM((_CPW,), jnp.float32),
            pltpu.VMEM((_CPW,), jnp.int32),
            pltpu.SemaphoreType.DMA,
            pltpu.SemaphoreType.DMA,
        ],
    )


def kernel(x):
    thr4, neq4 = pl.pallas_call(
        _tc_threshold_body,
        grid=(_B, _C // 128),
        in_specs=[pl.BlockSpec((1, _S, 128), lambda b, cb: (b, 0, cb))],
        out_specs=[
            pl.BlockSpec((1, 1, 1, 128), lambda b, cb: (b, cb, 0, 0)),
            pl.BlockSpec((1, 1, 1, 128), lambda b, cb: (b, cb, 0, 0)),
        ],
        out_shape=[
            jax.ShapeDtypeStruct((_B, _C // 128, 1, 128), jnp.float32),
            jax.ShapeDtypeStruct((_B, _C // 128, 1, 128), jnp.int32),
        ],
    )(x)
    thr = thr4.reshape(_B * _C)
    neq = neq4.reshape(_B * _C)
    out_flat = _sc_select()(x, thr, neq)
    out = out_flat.reshape(_B, _NW // _B, _CPW, _K)
    return out.transpose(0, 3, 1, 2).reshape(_B, _K, _C)


# speedup vs baseline: 1.0741x; 1.0741x over previous
import functools

import jax
import jax.numpy as jnp
from jax import lax
from jax.experimental import pallas as pl
from jax.experimental.pallas import tpu as pltpu
from jax.experimental.pallas import tpu_sc as plsc

_K = 64
_B = 4
_S = 4096
_C = 1024
_MININT = -(2**31)

_NC = 2
_NS = 16
_NW = _NC * _NS
_CPW = _C // (_NW // _B)
_GRP = _CPW // 16


def _tc_threshold_body(x_ref, thr_ref, neq_ref):
    x = x_ref[0]
    b = lax.bitcast_convert_type(x, jnp.int32)
    s = jnp.bitwise_xor(
        b, lax.shift_right_logical(jnp.right_shift(b, 31), jnp.int32(1)))

    hi = jnp.right_shift(s, 16).astype(jnp.int16).reshape(64, 64, 128)
    l16 = ((s & 0xFFFF) - 32768).astype(jnp.int16).reshape(64, 64, 128)

    def _cnt16(mask):
        vals = [mask[a].astype(jnp.int16) for a in range(64)]
        while len(vals) > 1:
            vals = [vals[i] + vals[i + 1] for i in range(0, len(vals), 2)]
        return jnp.sum(vals[0].astype(jnp.int32), axis=0, keepdims=True)

    def _descend(data, k_tgt):
        def bit_step(i, ph):
            bit = jnp.left_shift(jnp.int32(1), 15 - i)
            cand = jnp.bitwise_or(ph, bit)
            cand_s = (cand - 32768).astype(jnp.int16).reshape(1, 1, 128)
            cnt = _cnt16(data >= cand_s)
            return jnp.where(cnt >= k_tgt, cand, ph)

        return lax.fori_loop(0, 16, bit_step, jnp.zeros((1, 128), jnp.int32))

    ph = _descend(hi, _K)
    thi_s = (ph - 32768).astype(jnp.int16).reshape(1, 1, 128)
    n_hi_gt = _cnt16(hi > thi_s)
    k2 = _K - n_hi_gt
    loe = jnp.where(hi == thi_s, l16, jnp.int16(-32768))
    pl_ = _descend(loe, k2)

    t_s = jnp.left_shift(ph - 32768, 16) | pl_
    tlo_s = (pl_ - 32768).astype(jnp.int16).reshape(1, 1, 128)
    n_gt = n_hi_gt + _cnt16(loe > tlo_s)
    b_thr = jnp.where(t_s < 0, jnp.bitwise_not(jnp.bitwise_xor(t_s, _MININT)), t_s)
    thr_ref[...] = lax.bitcast_convert_type(b_thr, jnp.float32)[None, None]
    neq_ref[...] = (_K - n_gt)[None, None]


_CHUNK = 256


def _sc_select_body(x_hbm, thr_hbm, neq_hbm, out_hbm, xbufa, xbufb, obufs,
                    tbuf, nbuf, sema, semb):
    wid = lax.axis_index("c") * _NS + lax.axis_index("s")
    b = wid // (_NW // _B)
    cbase = (wid % (_NW // _B)) * _CPW
    coff = b * _C + cbase
    lane = lax.iota(jnp.int32, 16)

    pltpu.sync_copy(thr_hbm.at[pl.ds(coff, _CPW)], tbuf)
    pltpu.sync_copy(neq_hbm.at[pl.ds(coff, _CPW)], nbuf)

    thrs = [tbuf[pl.ds(g * 16, 16)] for g in range(_GRP)]
    st0 = tuple(
        [lane * _K for _ in range(_GRP)]
        + [nbuf[pl.ds(g * 16, 16)] for g in range(_GRP)])

    def _copy(ch, buf, sem):
        return pltpu.make_async_copy(
            x_hbm.at[b, pl.ds(ch * _CHUNK, _CHUNK), pl.ds(cbase, _CPW)],
            buf, sem)

    def _process(xbuf, st):
        @plsc.parallel_loop(0, _CHUNK, unroll=4, carry=st)
        def step(i, s2):
            vs = [xbuf[i, pl.ds(g * 16, 16)] for g in range(_GRP)]
            addrs, rems = list(s2[:_GRP]), list(s2[_GRP:])
            for g in range(_GRP):
                gt = vs[g] > thrs[g]
                eq = vs[g] == thrs[g]
                sel = jnp.logical_or(gt, jnp.logical_and(eq, rems[g] > 0))
                plsc.store_scatter(obufs[g], [addrs[g]], vs[g], mask=sel)
                rems[g] = rems[g] - eq.astype(jnp.int32)
                addrs[g] = addrs[g] + sel.astype(jnp.int32)
            return tuple(addrs + rems)

        return step

    npairs = _S // _CHUNK // 2
    _copy(0, xbufa, sema).start()

    def pair(p, st):
        _copy(2 * p, xbufa, sema).wait()
        _copy(2 * p + 1, xbufb, semb).start()
        st = _process(xbufa, st)
        _copy(2 * p + 1, xbufb, semb).wait()

        @pl.when(p < npairs - 1)
        def _():
            _copy(2 * p + 2, xbufa, sema).start()

        return _process(xbufb, st)

    lax.fori_loop(0, npairs, pair, st0)
    for g in range(_GRP):
        pltpu.sync_copy(
            obufs[g],
            out_hbm.at[pl.ds(wid * (_CPW * _K) + g * 16 * _K, 16 * _K)])


@functools.cache
def _sc_select():
    return pl.kernel(
        _sc_select_body,
        out_type=jax.ShapeDtypeStruct((_NW * _CPW * _K,), jnp.float32),
        mesh=plsc.VectorSubcoreMesh(
            core_axis_name="c", subcore_axis_name="s",
            num_cores=_NC, num_subcores=_NS,
        ),
        compiler_params=pltpu.CompilerParams(needs_layout_passes=False),
        scratch_types=[
            pltpu.VMEM((_CHUNK, _CPW), jnp.float32),
            pltpu.VMEM((_CHUNK, _CPW), jnp.float32),
            [pltpu.VMEM((16 * _K,), jnp.float32) for _ in range(_GRP)],
            pltpu.VMEM((_CPW,), jnp.float32),
            pltpu.VMEM((_CPW,), jnp.int32),
            pltpu.SemaphoreType.DMA,
            pltpu.SemaphoreType.DMA,
        ],
    )


def kernel(x):
    thr4, neq4 = pl.pallas_call(
        _tc_threshold_body,
        grid=(_B, _C // 128),
        in_specs=[pl.BlockSpec((1, _S, 128), lambda b, cb: (b, 0, cb))],
        out_specs=[
            pl.BlockSpec((1, 1, 1, 128), lambda b, cb: (b, cb, 0, 0)),
            pl.BlockSpec((1, 1, 1, 128), lambda b, cb: (b, cb, 0, 0)),
        ],
        out_shape=[
            jax.ShapeDtypeStruct((_B, _C // 128, 1, 128), jnp.float32),
            jax.ShapeDtypeStruct((_B, _C // 128, 1, 128), jnp.int32),
        ],
    )(x)
    thr = thr4.reshape(_B * _C)
    neq = neq4.reshape(_B * _C)
    out_flat = _sc_select()(x, thr, neq)
    out = out_flat.reshape(_B, _NW // _B, _CPW, _K)
    return out.transpose(0, 3, 1, 2).reshape(_B, _K, _C)


# speedup vs baseline: 1.0746x; 1.0005x over previous
import functools

import jax
import jax.numpy as jnp
from jax import lax
from jax.experimental import pallas as pl
from jax.experimental.pallas import tpu as pltpu
from jax.experimental.pallas import tpu_sc as plsc

_K = 64
_B = 4
_S = 4096
_C = 1024
_MININT = -(2**31)

_NC = 2
_NS = 16
_NW = _NC * _NS
_CPW = _C // (_NW // _B)
_TCW = 256
_GRP = _CPW // 16


def _tc_threshold_body(x_ref, thr_ref, neq_ref):
    x = x_ref[0]
    b = lax.bitcast_convert_type(x, jnp.int32)
    s = jnp.bitwise_xor(
        b, lax.shift_right_logical(jnp.right_shift(b, 31), jnp.int32(1)))

    hi = jnp.right_shift(s, 16).astype(jnp.int16).reshape(64, 64, _TCW)
    l16 = ((s & 0xFFFF) - 32768).astype(jnp.int16).reshape(64, 64, _TCW)

    def _cnt16(mask):
        vals = [mask[a].astype(jnp.int16) for a in range(64)]
        while len(vals) > 1:
            vals = [vals[i] + vals[i + 1] for i in range(0, len(vals), 2)]
        return jnp.sum(vals[0].astype(jnp.int32), axis=0, keepdims=True)

    def _descend(data, k_tgt):
        def bit_step(i, ph):
            bit = jnp.left_shift(jnp.int32(1), 15 - i)
            cand = jnp.bitwise_or(ph, bit)
            cand_s = (cand - 32768).astype(jnp.int16).reshape(1, 1, _TCW)
            cnt = _cnt16(data >= cand_s)
            return jnp.where(cnt >= k_tgt, cand, ph)

        return lax.fori_loop(0, 16, bit_step, jnp.zeros((1, _TCW), jnp.int32))

    ph = _descend(hi, _K)
    thi_s = (ph - 32768).astype(jnp.int16).reshape(1, 1, _TCW)
    n_hi_gt = _cnt16(hi > thi_s)
    k2 = _K - n_hi_gt
    loe = jnp.where(hi == thi_s, l16, jnp.int16(-32768))
    pl_ = _descend(loe, k2)

    t_s = jnp.left_shift(ph - 32768, 16) | pl_
    tlo_s = (pl_ - 32768).astype(jnp.int16).reshape(1, 1, _TCW)
    n_gt = n_hi_gt + _cnt16(loe > tlo_s)
    b_thr = jnp.where(t_s < 0, jnp.bitwise_not(jnp.bitwise_xor(t_s, _MININT)), t_s)
    thr_ref[...] = lax.bitcast_convert_type(b_thr, jnp.float32)[None, None]
    neq_ref[...] = (_K - n_gt)[None, None]


_CHUNK = 256


def _sc_select_body(x_hbm, thr_hbm, neq_hbm, out_hbm, xbufa, xbufb, obufs,
                    tbuf, nbuf, sema, semb):
    wid = lax.axis_index("c") * _NS + lax.axis_index("s")
    b = wid // (_NW // _B)
    cbase = (wid % (_NW // _B)) * _CPW
    coff = b * _C + cbase
    lane = lax.iota(jnp.int32, 16)

    pltpu.sync_copy(thr_hbm.at[pl.ds(coff, _CPW)], tbuf)
    pltpu.sync_copy(neq_hbm.at[pl.ds(coff, _CPW)], nbuf)

    thrs = [tbuf[pl.ds(g * 16, 16)] for g in range(_GRP)]
    st0 = tuple(
        [lane * _K for _ in range(_GRP)]
        + [nbuf[pl.ds(g * 16, 16)] for g in range(_GRP)])

    def _copy(ch, buf, sem):
        return pltpu.make_async_copy(
            x_hbm.at[b, pl.ds(ch * _CHUNK, _CHUNK), pl.ds(cbase, _CPW)],
            buf, sem)

    def _process(xbuf, st):
        @plsc.parallel_loop(0, _CHUNK, unroll=4, carry=st)
        def step(i, s2):
            vs = [xbuf[i, pl.ds(g * 16, 16)] for g in range(_GRP)]
            addrs, rems = list(s2[:_GRP]), list(s2[_GRP:])
            for g in range(_GRP):
                gt = vs[g] > thrs[g]
                eq = vs[g] == thrs[g]
                sel = jnp.logical_or(gt, jnp.logical_and(eq, rems[g] > 0))
                plsc.store_scatter(obufs[g], [addrs[g]], vs[g], mask=sel)
                rems[g] = rems[g] - eq.astype(jnp.int32)
                addrs[g] = addrs[g] + sel.astype(jnp.int32)
            return tuple(addrs + rems)

        return step

    npairs = _S // _CHUNK // 2
    _copy(0, xbufa, sema).start()

    def pair(p, st):
        _copy(2 * p, xbufa, sema).wait()
        _copy(2 * p + 1, xbufb, semb).start()
        st = _process(xbufa, st)
        _copy(2 * p + 1, xbufb, semb).wait()

        @pl.when(p < npairs - 1)
        def _():
            _copy(2 * p + 2, xbufa, sema).start()

        return _process(xbufb, st)

    lax.fori_loop(0, npairs, pair, st0)
    for g in range(_GRP):
        pltpu.sync_copy(
            obufs[g],
            out_hbm.at[pl.ds(wid * (_CPW * _K) + g * 16 * _K, 16 * _K)])


@functools.cache
def _sc_select():
    return pl.kernel(
        _sc_select_body,
        out_type=jax.ShapeDtypeStruct((_NW * _CPW * _K,), jnp.float32),
        mesh=plsc.VectorSubcoreMesh(
            core_axis_name="c", subcore_axis_name="s",
            num_cores=_NC, num_subcores=_NS,
        ),
        compiler_params=pltpu.CompilerParams(needs_layout_passes=False),
        scratch_types=[
            pltpu.VMEM((_CHUNK, _CPW), jnp.float32),
            pltpu.VMEM((_CHUNK, _CPW), jnp.float32),
            [pltpu.VMEM((16 * _K,), jnp.float32) for _ in range(_GRP)],
            pltpu.VMEM((_CPW,), jnp.float32),
            pltpu.VMEM((_CPW,), jnp.int32),
            pltpu.SemaphoreType.DMA,
            pltpu.SemaphoreType.DMA,
        ],
    )


def kernel(x):
    thr4, neq4 = pl.pallas_call(
        _tc_threshold_body,
        grid=(_B, _C // _TCW),
        in_specs=[pl.BlockSpec((1, _S, _TCW), lambda b, cb: (b, 0, cb))],
        out_specs=[
            pl.BlockSpec((1, 1, 1, _TCW), lambda b, cb: (b, cb, 0, 0)),
            pl.BlockSpec((1, 1, 1, _TCW), lambda b, cb: (b, cb, 0, 0)),
        ],
        out_shape=[
            jax.ShapeDtypeStruct((_B, _C // _TCW, 1, _TCW), jnp.float32),
            jax.ShapeDtypeStruct((_B, _C // _TCW, 1, _TCW), jnp.int32),
        ],
    )(x)
    thr = thr4.reshape(_B * _C)
    neq = neq4.reshape(_B * _C)
    out_flat = _sc_select()(x, thr, neq)
    out = out_flat.reshape(_B, _NW // _B, _CPW, _K)
    return out.transpose(0, 3, 1, 2).reshape(_B, _K, _C)
